# baseline (speedup 1.0000x reference)
import functools

import jax
import jax.numpy as jnp
from jax import lax
from jax.experimental import pallas as pl
from jax.experimental.pallas import tpu as pltpu
from jax.experimental.pallas import tpu_sc as plsc

_N_NODES = 10000
_N_EDGES = 320000
_D_NODE = 128
_D_EDGE = 16
_NS = 16
_L = 16
_EPW = _N_EDGES // _NS
_NPAD = 10240
_STRIPE = _NPAD // _NS
_CHUNK_E = 2000
_NCHUNK = _EPW // _CHUNK_E
_CFLAT = _CHUNK_E * _D_EDGE
_NROWS = 128
_NSUB = 5


def _sc_segment_sum(node_flat, edge_flat, gi_flat):
    mesh = plsc.VectorSubcoreMesh(
        core_axis_name="c", subcore_axis_name="s", num_cores=1)

    @functools.partial(
        pl.kernel,
        out_type=jax.ShapeDtypeStruct((_N_NODES,), jnp.float32),
        mesh=mesh,
        compiler_params=pltpu.CompilerParams(needs_layout_passes=False),
        scratch_types=[
            pltpu.VMEM((_NPAD,), jnp.float32),
            pltpu.VMEM((_CFLAT,), jnp.float32),
            pltpu.VMEM((_CFLAT,), jnp.float32),
            pltpu.VMEM((2 * _CHUNK_E,), jnp.int32),
            pltpu.VMEM((2 * _CHUNK_E,), jnp.int32),
            pltpu.VMEM((_NPAD,), jnp.float32),
            pltpu.VMEM((_NROWS,), jnp.float32),
            pltpu.VMEM((_STRIPE,), jnp.float32),
            pltpu.VMEM((_STRIPE,), jnp.float32),
            pltpu.VMEM_SHARED((_NPAD,), jnp.float32),
            pltpu.VMEM_SHARED((_NS * _NPAD,), jnp.float32),
            pltpu.SemaphoreType.DMA,
            pltpu.SemaphoreType.DMA,
            pltpu.SemaphoreType.DMA,
            pltpu.SemaphoreType.DMA,
        ],
    )
    def k(nodes_hbm, ef_hbm, gi_hbm, out_hbm,
          nodes_v, ec0, ec1, gc0, gc1, acc_v, stage_v, red_v, tmp_v,
          nodes_sh, acc_sh, sem0, sem1, sem2, sem3):
        s = lax.axis_index("s")
        base = s * _EPW

        iota = lax.iota(jnp.int32, _L)

        base_row = s * _STRIPE
        for sub in range(_NSUB):
            start = jnp.minimum(base_row + sub * _NROWS, _N_NODES - _NROWS)
            pltpu.sync_copy(
                nodes_hbm.at[pl.ds(start * _D_NODE, _NROWS * _D_NODE)],
                ec0.at[pl.ds(0, _NROWS * _D_NODE)])

            def ext_body(j, carry):
                idx = (j * _L + iota) * _D_NODE
                stage_v[pl.ds(j * _L, _L)] = plsc.load_gather(ec0, [idx])
                return carry

            lax.fori_loop(0, _NROWS // _L, ext_body, 0)
            pltpu.sync_copy(stage_v, nodes_sh.at[pl.ds(start, _NROWS)])

        zeros = jnp.zeros((_L,), jnp.float32)

        def zero_body(i, carry):
            acc_v[pl.ds(i * _L, _L)] = zeros
            return carry

        lax.fori_loop(0, _NPAD // _L, zero_body, 0)

        plsc.subcore_barrier()
        pltpu.sync_copy(nodes_sh, nodes_v)

        ebufs = (ec0, ec1)
        gbufs = (gc0, gc1)
        esems = (sem0, sem1)
        gsems = (sem2, sem3)
        flat_base = base * _D_EDGE
        ecps = [None, None]
        gcps = [None, None]
        ecps[0] = pltpu.async_copy(
            ef_hbm.at[pl.ds(flat_base, _CFLAT)], ebufs[0], esems[0])
        gcps[0] = pltpu.async_copy(
            gi_hbm.at[pl.ds(2 * base, 2 * _CHUNK_E)], gbufs[0], gsems[0])

        iota2 = iota * 2
        iota16 = iota * _D_EDGE

        for ci in range(_NCHUNK):
            if ci + 1 < _NCHUNK:
                nb = (ci + 1) % 2
                ecps[nb] = pltpu.async_copy(
                    ef_hbm.at[pl.ds(flat_base + (ci + 1) * _CFLAT, _CFLAT)],
                    ebufs[nb], esems[nb])
                gcps[nb] = pltpu.async_copy(
                    gi_hbm.at[pl.ds(2 * (base + (ci + 1) * _CHUNK_E),
                                    2 * _CHUNK_E)],
                    gbufs[nb], gsems[nb])
            ecps[ci % 2].wait()
            gcps[ci % 2].wait()
            ebuf = ebufs[ci % 2]
            gbuf = gbufs[ci % 2]

            def body(j, carry, ebuf=ebuf, gbuf=gbuf):
                flat = 2 * (j * _L) + iota2
                s_idx = plsc.load_gather(gbuf, [flat])
                r_idx = plsc.load_gather(gbuf, [flat + 1])
                e = plsc.load_gather(ebuf, [j * (_L * _D_EDGE) + iota16])
                ns = plsc.load_gather(nodes_v, [s_idx])
                nr = plsc.load_gather(nodes_v, [r_idx])
                upd = (ns - nr) / e
                plsc.addupdate_scatter(acc_v, [r_idx], upd)
                return carry

            lax.fori_loop(0, _CHUNK_E // _L, body, 0)

        pltpu.sync_copy(acc_v, acc_sh.at[pl.ds(s * _NPAD, _NPAD)])
        plsc.subcore_barrier()

        stripe = s * _STRIPE

        def rsum_body(i, carry):
            off = i * _L
            red_v[pl.ds(off, _L)] = (
                red_v[pl.ds(off, _L)] + tmp_v[pl.ds(off, _L)])
            return carry

        pltpu.sync_copy(acc_sh.at[pl.ds(stripe, _STRIPE)], red_v)
        for t in range(1, _NS):
            pltpu.sync_copy(
                acc_sh.at[pl.ds(t * _NPAD + stripe, _STRIPE)], tmp_v)
            lax.fori_loop(0, _STRIPE // _L, rsum_body, 0)

        @pl.when(s < _NS - 1)
        def _():
            pltpu.sync_copy(red_v, out_hbm.at[pl.ds(stripe, _STRIPE)])

        @pl.when(s == _NS - 1)
        def _():
            pltpu.sync_copy(
                red_v.at[pl.ds(0, _N_NODES - (_NS - 1) * _STRIPE)],
                out_hbm.at[pl.ds(stripe, _N_NODES - (_NS - 1) * _STRIPE)])

    return k(node_flat, edge_flat, gi_flat)


def kernel(input_node, input_edge, graph_index):
    return _sc_segment_sum(
        input_node.reshape(-1), input_edge.reshape(-1),
        graph_index.reshape(-1))


# speedup vs baseline: 1.0113x; 1.0113x over previous
import functools

import jax
import jax.numpy as jnp
from jax import lax
from jax.experimental import pallas as pl
from jax.experimental.pallas import tpu as pltpu
from jax.experimental.pallas import tpu_sc as plsc

_N_NODES = 10000
_N_EDGES = 320000
_D_NODE = 128
_D_EDGE = 16
_NS = 16
_L = 16
_EPW = _N_EDGES // _NS
_NPAD = 10240
_STRIPE = _NPAD // _NS
_CHUNK_E = 2000
_NCHUNK = _EPW // _CHUNK_E
_CFLAT = _CHUNK_E * _D_EDGE
_NROWS = 128
_NSUB = 5
_U = 5


def _sc_segment_sum(node_flat, edge_flat, gi_flat):
    mesh = plsc.VectorSubcoreMesh(
        core_axis_name="c", subcore_axis_name="s", num_cores=1)

    @functools.partial(
        pl.kernel,
        out_type=jax.ShapeDtypeStruct((_N_NODES,), jnp.float32),
        mesh=mesh,
        compiler_params=pltpu.CompilerParams(needs_layout_passes=False),
        scratch_types=[
            pltpu.VMEM((_NPAD,), jnp.float32),
            pltpu.VMEM((_CFLAT,), jnp.float32),
            pltpu.VMEM((_CFLAT,), jnp.float32),
            pltpu.VMEM((2 * _CHUNK_E,), jnp.int32),
            pltpu.VMEM((2 * _CHUNK_E,), jnp.int32),
            pltpu.VMEM((_NPAD,), jnp.float32),
            pltpu.VMEM((_NROWS,), jnp.float32),
            pltpu.VMEM((_STRIPE,), jnp.float32),
            pltpu.VMEM((_STRIPE,), jnp.float32),
            pltpu.VMEM_SHARED((_NPAD,), jnp.float32),
            pltpu.VMEM_SHARED((_NS * _NPAD,), jnp.float32),
            pltpu.SemaphoreType.DMA,
            pltpu.SemaphoreType.DMA,
            pltpu.SemaphoreType.DMA,
            pltpu.SemaphoreType.DMA,
        ],
    )
    def k(nodes_hbm, ef_hbm, gi_hbm, out_hbm,
          nodes_v, ec0, ec1, gc0, gc1, acc_v, stage_v, red_v, tmp_v,
          nodes_sh, acc_sh, sem0, sem1, sem2, sem3):
        s = lax.axis_index("s")
        base = s * _EPW

        iota = lax.iota(jnp.int32, _L)

        base_row = s * _STRIPE
        for sub in range(_NSUB):
            start = jnp.minimum(base_row + sub * _NROWS, _N_NODES - _NROWS)
            pltpu.sync_copy(
                nodes_hbm.at[pl.ds(start * _D_NODE, _NROWS * _D_NODE)],
                ec0.at[pl.ds(0, _NROWS * _D_NODE)])

            def ext_body(j, carry):
                idx = (j * _L + iota) * _D_NODE
                stage_v[pl.ds(j * _L, _L)] = plsc.load_gather(ec0, [idx])
                return carry

            lax.fori_loop(0, _NROWS // _L, ext_body, 0)
            pltpu.sync_copy(stage_v, nodes_sh.at[pl.ds(start, _NROWS)])

        zeros = jnp.zeros((_L,), jnp.float32)

        def zero_body(i, carry):
            acc_v[pl.ds(i * _L, _L)] = zeros
            return carry

        lax.fori_loop(0, _NPAD // _L, zero_body, 0)

        plsc.subcore_barrier()
        pltpu.sync_copy(nodes_sh, nodes_v)

        ebufs = (ec0, ec1)
        gbufs = (gc0, gc1)
        esems = (sem0, sem1)
        gsems = (sem2, sem3)
        flat_base = base * _D_EDGE
        ecps = [None, None]
        gcps = [None, None]
        ecps[0] = pltpu.async_copy(
            ef_hbm.at[pl.ds(flat_base, _CFLAT)], ebufs[0], esems[0])
        gcps[0] = pltpu.async_copy(
            gi_hbm.at[pl.ds(2 * base, 2 * _CHUNK_E)], gbufs[0], gsems[0])

        iota2 = iota * 2
        iota16 = iota * _D_EDGE

        for ci in range(_NCHUNK):
            if ci + 1 < _NCHUNK:
                nb = (ci + 1) % 2
                ecps[nb] = pltpu.async_copy(
                    ef_hbm.at[pl.ds(flat_base + (ci + 1) * _CFLAT, _CFLAT)],
                    ebufs[nb], esems[nb])
                gcps[nb] = pltpu.async_copy(
                    gi_hbm.at[pl.ds(2 * (base + (ci + 1) * _CHUNK_E),
                                    2 * _CHUNK_E)],
                    gbufs[nb], gsems[nb])
            ecps[ci % 2].wait()
            gcps[ci % 2].wait()
            ebuf = ebufs[ci % 2]
            gbuf = gbufs[ci % 2]

            def body(j, carry, ebuf=ebuf, gbuf=gbuf):
                gath = []
                for g in range(_U):
                    off = j * (_U * _L) + g * _L
                    flat = 2 * off + iota2
                    s_idx = plsc.load_gather(gbuf, [flat])
                    r_idx = plsc.load_gather(gbuf, [flat + 1])
                    e = plsc.load_gather(ebuf, [off * _D_EDGE + iota16])
                    gath.append((s_idx, r_idx, e))
                for s_idx, r_idx, e in gath:
                    ns = plsc.load_gather(nodes_v, [s_idx])
                    nr = plsc.load_gather(nodes_v, [r_idx])
                    upd = (ns - nr) / e
                    plsc.addupdate_scatter(acc_v, [r_idx], upd)
                return carry

            lax.fori_loop(0, _CHUNK_E // (_U * _L), body, 0)

        pltpu.sync_copy(acc_v, acc_sh.at[pl.ds(s * _NPAD, _NPAD)])
        plsc.subcore_barrier()

        stripe = s * _STRIPE

        def rsum_body(i, carry):
            off = i * _L
            red_v[pl.ds(off, _L)] = (
                red_v[pl.ds(off, _L)] + tmp_v[pl.ds(off, _L)])
            return carry

        pltpu.sync_copy(acc_sh.at[pl.ds(stripe, _STRIPE)], red_v)
        for t in range(1, _NS):
            pltpu.sync_copy(
                acc_sh.at[pl.ds(t * _NPAD + stripe, _STRIPE)], tmp_v)
            lax.fori_loop(0, _STRIPE // _L, rsum_body, 0)

        @pl.when(s < _NS - 1)
        def _():
            pltpu.sync_copy(red_v, out_hbm.at[pl.ds(stripe, _STRIPE)])

        @pl.when(s == _NS - 1)
        def _():
            pltpu.sync_copy(
                red_v.at[pl.ds(0, _N_NODES - (_NS - 1) * _STRIPE)],
                out_hbm.at[pl.ds(stripe, _N_NODES - (_NS - 1) * _STRIPE)])

    return k(node_flat, edge_flat, gi_flat)


def kernel(input_node, input_edge, graph_index):
    return _sc_segment_sum(
        input_node.reshape(-1), input_edge.reshape(-1),
        graph_index.reshape(-1))


# speedup vs baseline: 6.7003x; 6.6251x over previous
import functools

import jax
import jax.numpy as jnp
from jax import lax
from jax.experimental import pallas as pl
from jax.experimental.pallas import tpu as pltpu
from jax.experimental.pallas import tpu_sc as plsc

_N_NODES = 10000
_N_EDGES = 320000
_D_NODE = 128
_D_EDGE = 16
_NS = 16
_L = 16
_EPW = _N_EDGES // _NS
_NPAD = 10240
_STRIPE = _NPAD // _NS
_CHUNK_E = 2000
_NCHUNK = _EPW // _CHUNK_E
_WIN = 2176
_NROWS = 128
_NSUB = 5
_U = 5


def _sc_segment_sum(nodes_2d, edges_t, gi_t):
    mesh = plsc.VectorSubcoreMesh(
        core_axis_name="c", subcore_axis_name="s", num_cores=1)

    @functools.partial(
        pl.kernel,
        out_type=jax.ShapeDtypeStruct((_N_NODES,), jnp.float32),
        mesh=mesh,
        compiler_params=pltpu.CompilerParams(needs_layout_passes=False),
        scratch_types=[
            pltpu.VMEM((_NPAD,), jnp.float32),
            pltpu.VMEM((_NROWS, _D_NODE), jnp.float32),
            pltpu.VMEM((8, _WIN), jnp.float32),
            pltpu.VMEM((8, _WIN), jnp.float32),
            pltpu.VMEM((2, _WIN), jnp.int32),
            pltpu.VMEM((2, _WIN), jnp.int32),
            pltpu.VMEM((_NPAD,), jnp.float32),
            pltpu.VMEM((_NROWS,), jnp.float32),
            pltpu.VMEM((_STRIPE,), jnp.float32),
            pltpu.VMEM((_STRIPE,), jnp.float32),
            pltpu.VMEM_SHARED((_NPAD,), jnp.float32),
            pltpu.VMEM_SHARED((_NS * _NPAD,), jnp.float32),
            pltpu.SemaphoreType.DMA,
            pltpu.SemaphoreType.DMA,
            pltpu.SemaphoreType.DMA,
            pltpu.SemaphoreType.DMA,
        ],
    )
    def k(nodes_hbm, ef_hbm, gi_hbm, out_hbm,
          nodes_v, rows_v, ec0, ec1, gc0, gc1, acc_v, stage_v, red_v, tmp_v,
          nodes_sh, acc_sh, sem0, sem1, sem2, sem3):
        s = lax.axis_index("s")
        base = s * _EPW

        iota = lax.iota(jnp.int32, _L)
        col0 = jnp.zeros((_L,), jnp.int32)
        row0 = col0
        row1 = jnp.ones((_L,), jnp.int32)

        base_row = s * _STRIPE
        for sub in range(_NSUB):
            start = jnp.minimum(base_row + sub * _NROWS, _N_NODES - _NROWS)
            pltpu.sync_copy(nodes_hbm.at[pl.ds(start, _NROWS)], rows_v)

            def ext_body(j, carry):
                stage_v[pl.ds(j * _L, _L)] = plsc.load_gather(
                    rows_v, [j * _L + iota, col0])
                return carry

            lax.fori_loop(0, _NROWS // _L, ext_body, 0)
            pltpu.sync_copy(stage_v, nodes_sh.at[pl.ds(start, _NROWS)])

        zeros = jnp.zeros((_L,), jnp.float32)

        def zero_body(i, carry):
            acc_v[pl.ds(i * _L, _L)] = zeros
            return carry

        lax.fori_loop(0, _NPAD // _L, zero_body, 0)

        plsc.subcore_barrier()
        pltpu.sync_copy(nodes_sh, nodes_v)

        ebufs = (ec0, ec1)
        gbufs = (gc0, gc1)
        esems = (sem0, sem1)
        gsems = (sem2, sem3)

        def win_start(ci):
            cbase = base + ci * _CHUNK_E
            w = jnp.minimum((cbase // 128) * 128, _N_EDGES - _WIN)
            w = pl.multiple_of(w, 128)
            return w, cbase - w

        ecps = [None, None]
        gcps = [None, None]
        w0, _ = win_start(0)
        ecps[0] = pltpu.async_copy(
            ef_hbm.at[pl.ds(0, 8), pl.ds(w0, _WIN)], ebufs[0], esems[0])
        gcps[0] = pltpu.async_copy(
            gi_hbm.at[:, pl.ds(w0, _WIN)], gbufs[0], gsems[0])

        for ci in range(_NCHUNK):
            if ci + 1 < _NCHUNK:
                nb = (ci + 1) % 2
                wn, _ = win_start(ci + 1)
                ecps[nb] = pltpu.async_copy(
                    ef_hbm.at[pl.ds(0, 8), pl.ds(wn, _WIN)],
                    ebufs[nb], esems[nb])
                gcps[nb] = pltpu.async_copy(
                    gi_hbm.at[:, pl.ds(wn, _WIN)], gbufs[nb], gsems[nb])
            ecps[ci % 2].wait()
            gcps[ci % 2].wait()
            ebuf = ebufs[ci % 2]
            gbuf = gbufs[ci % 2]
            _, rj = win_start(ci)

            def body(j, carry, ebuf=ebuf, gbuf=gbuf, rj=rj):
                gath = []
                for g in range(_U):
                    pos = rj + j * (_U * _L) + g * _L + iota
                    s_idx = plsc.load_gather(gbuf, [row0, pos])
                    r_idx = plsc.load_gather(gbuf, [row1, pos])
                    e = plsc.load_gather(ebuf, [row0, pos])
                    gath.append((s_idx, r_idx, e))
                for s_idx, r_idx, e in gath:
                    ns = plsc.load_gather(nodes_v, [s_idx])
                    nr = plsc.load_gather(nodes_v, [r_idx])
                    upd = (ns - nr) / e
                    plsc.addupdate_scatter(acc_v, [r_idx], upd)
                return carry

            lax.fori_loop(0, _CHUNK_E // (_U * _L), body, 0)

        pltpu.sync_copy(acc_v, acc_sh.at[pl.ds(s * _NPAD, _NPAD)])
        plsc.subcore_barrier()

        stripe = s * _STRIPE

        def rsum_body(i, carry):
            off = i * _L
            red_v[pl.ds(off, _L)] = (
                red_v[pl.ds(off, _L)] + tmp_v[pl.ds(off, _L)])
            return carry

        pltpu.sync_copy(acc_sh.at[pl.ds(stripe, _STRIPE)], red_v)
        for t in range(1, _NS):
            pltpu.sync_copy(
                acc_sh.at[pl.ds(t * _NPAD + stripe, _STRIPE)], tmp_v)
            lax.fori_loop(0, _STRIPE // _L, rsum_body, 0)

        @pl.when(s < _NS - 1)
        def _():
            pltpu.sync_copy(red_v, out_hbm.at[pl.ds(stripe, _STRIPE)])

        @pl.when(s == _NS - 1)
        def _():
            pltpu.sync_copy(
                red_v.at[pl.ds(0, _N_NODES - (_NS - 1) * _STRIPE)],
                out_hbm.at[pl.ds(stripe, _N_NODES - (_NS - 1) * _STRIPE)])

    return k(nodes_2d, edges_t, gi_t)


def kernel(input_node, input_edge, graph_index):
    return _sc_segment_sum(input_node, input_edge.T, graph_index.T)


# speedup vs baseline: 7.3862x; 1.1024x over previous
import functools

import jax
import jax.numpy as jnp
from jax import lax
from jax.experimental import pallas as pl
from jax.experimental.pallas import tpu as pltpu
from jax.experimental.pallas import tpu_sc as plsc

_N_NODES = 10000
_N_EDGES = 320000
_D_NODE = 128
_D_EDGE = 16
_NC = 2
_NS = 16
_NW = _NC * _NS
_L = 16
_EPW = _N_EDGES // _NW
_NPAD = 10240
_STRIPE = _NPAD // _NS
_CHUNK_E = 2000
_NCHUNK = _EPW // _CHUNK_E
_WIN = 2176
_NROWS = 128
_NSUB = 5
_U = 5


def _sc_segment_sum(nodes_2d, edges_t, gi_t):
    mesh = plsc.VectorSubcoreMesh(core_axis_name="c", subcore_axis_name="s")

    @functools.partial(
        pl.kernel,
        out_type=jax.ShapeDtypeStruct((_NC * _NPAD,), jnp.float32),
        mesh=mesh,
        compiler_params=pltpu.CompilerParams(needs_layout_passes=False),
        scratch_types=[
            pltpu.VMEM((_NPAD,), jnp.float32),
            pltpu.VMEM((_NROWS, _D_NODE), jnp.float32),
            pltpu.VMEM((8, _WIN), jnp.float32),
            pltpu.VMEM((8, _WIN), jnp.float32),
            pltpu.VMEM((2, _WIN), jnp.int32),
            pltpu.VMEM((2, _WIN), jnp.int32),
            pltpu.VMEM((_NPAD,), jnp.float32),
            pltpu.VMEM((_NROWS,), jnp.float32),
            pltpu.VMEM((_STRIPE,), jnp.float32),
            pltpu.VMEM((_STRIPE,), jnp.float32),
            pltpu.VMEM_SHARED((_NPAD,), jnp.float32),
            pltpu.VMEM_SHARED((_NS * _NPAD,), jnp.float32),
            pltpu.SemaphoreType.DMA,
            pltpu.SemaphoreType.DMA,
            pltpu.SemaphoreType.DMA,
            pltpu.SemaphoreType.DMA,
        ],
    )
    def k(nodes_hbm, ef_hbm, gi_hbm, out_hbm,
          nodes_v, rows_v, ec0, ec1, gc0, gc1, acc_v, stage_v, red_v, tmp_v,
          nodes_sh, acc_sh, sem0, sem1, sem2, sem3):
        c = lax.axis_index("c")
        s = lax.axis_index("s")
        wid = s * _NC + c
        base = wid * _EPW

        iota = lax.iota(jnp.int32, _L)
        col0 = jnp.zeros((_L,), jnp.int32)
        row0 = col0
        row1 = jnp.ones((_L,), jnp.int32)

        base_row = s * _STRIPE
        for sub in range(_NSUB):
            start = jnp.minimum(base_row + sub * _NROWS, _N_NODES - _NROWS)
            pltpu.sync_copy(nodes_hbm.at[pl.ds(start, _NROWS)], rows_v)

            def ext_body(j, carry):
                stage_v[pl.ds(j * _L, _L)] = plsc.load_gather(
                    rows_v, [j * _L + iota, col0])
                return carry

            lax.fori_loop(0, _NROWS // _L, ext_body, 0)
            pltpu.sync_copy(stage_v, nodes_sh.at[pl.ds(start, _NROWS)])

        zeros = jnp.zeros((_L,), jnp.float32)

        def zero_body(i, carry):
            acc_v[pl.ds(i * _L, _L)] = zeros
            return carry

        lax.fori_loop(0, _NPAD // _L, zero_body, 0)

        plsc.subcore_barrier()
        pltpu.sync_copy(nodes_sh, nodes_v)

        ebufs = (ec0, ec1)
        gbufs = (gc0, gc1)
        esems = (sem0, sem1)
        gsems = (sem2, sem3)

        def win_start(ci):
            cbase = base + ci * _CHUNK_E
            w = jnp.minimum((cbase // 128) * 128, _N_EDGES - _WIN)
            w = pl.multiple_of(w, 128)
            return w, cbase - w

        ecps = [None, None]
        gcps = [None, None]
        w0, _ = win_start(0)
        ecps[0] = pltpu.async_copy(
            ef_hbm.at[pl.ds(0, 8), pl.ds(w0, _WIN)], ebufs[0], esems[0])
        gcps[0] = pltpu.async_copy(
            gi_hbm.at[:, pl.ds(w0, _WIN)], gbufs[0], gsems[0])

        for ci in range(_NCHUNK):
            if ci + 1 < _NCHUNK:
                nb = (ci + 1) % 2
                wn, _ = win_start(ci + 1)
                ecps[nb] = pltpu.async_copy(
                    ef_hbm.at[pl.ds(0, 8), pl.ds(wn, _WIN)],
                    ebufs[nb], esems[nb])
                gcps[nb] = pltpu.async_copy(
                    gi_hbm.at[:, pl.ds(wn, _WIN)], gbufs[nb], gsems[nb])
            ecps[ci % 2].wait()
            gcps[ci % 2].wait()
            ebuf = ebufs[ci % 2]
            gbuf = gbufs[ci % 2]
            _, rj = win_start(ci)

            def body(j, carry, ebuf=ebuf, gbuf=gbuf, rj=rj):
                gath = []
                for g in range(_U):
                    pos = rj + j * (_U * _L) + g * _L + iota
                    s_idx = plsc.load_gather(gbuf, [row0, pos])
                    r_idx = plsc.load_gather(gbuf, [row1, pos])
                    e = plsc.load_gather(ebuf, [row0, pos])
                    gath.append((s_idx, r_idx, e))
                for s_idx, r_idx, e in gath:
                    ns = plsc.load_gather(nodes_v, [s_idx])
                    nr = plsc.load_gather(nodes_v, [r_idx])
                    upd = (ns - nr) / e
                    plsc.addupdate_scatter(acc_v, [r_idx], upd)
                return carry

            lax.fori_loop(0, _CHUNK_E // (_U * _L), body, 0)

        pltpu.sync_copy(acc_v, acc_sh.at[pl.ds(s * _NPAD, _NPAD)])
        plsc.subcore_barrier()

        stripe = s * _STRIPE

        def rsum_body(i, carry):
            off = i * _L
            red_v[pl.ds(off, _L)] = (
                red_v[pl.ds(off, _L)] + tmp_v[pl.ds(off, _L)])
            return carry

        pltpu.sync_copy(acc_sh.at[pl.ds(stripe, _STRIPE)], red_v)
        for t in range(1, _NS):
            pltpu.sync_copy(
                acc_sh.at[pl.ds(t * _NPAD + stripe, _STRIPE)], tmp_v)
            lax.fori_loop(0, _STRIPE // _L, rsum_body, 0)

        pltpu.sync_copy(red_v, out_hbm.at[pl.ds(c * _NPAD + stripe, _STRIPE)])

    return k(nodes_2d, edges_t, gi_t)


def _tc_combine(partials):
    def body(p_ref, o_ref):
        a = p_ref[pl.ds(0, _NPAD)]
        b = p_ref[pl.ds(_NPAD, _NPAD)]
        o_ref[...] = (a + b)[: _N_NODES]

    return pl.pallas_call(
        body,
        out_shape=jax.ShapeDtypeStruct((_N_NODES,), jnp.float32),
    )(partials)


def kernel(input_node, input_edge, graph_index):
    partials = _sc_segment_sum(input_node, input_edge.T, graph_index.T)
    return _tc_combine(partials)


# speedup vs baseline: 8.1067x; 1.0975x over previous
import functools

import jax
import jax.numpy as jnp
from jax import lax
from jax.experimental import pallas as pl
from jax.experimental.pallas import tpu as pltpu
from jax.experimental.pallas import tpu_sc as plsc

_N_NODES = 10000
_N_EDGES = 320000
_D_NODE = 128
_D_EDGE = 16
_NC = 2
_NS = 16
_NW = _NC * _NS
_L = 16
_EPW = _N_EDGES // _NW
_NPAD = 10240
_STRIPE = _NPAD // _NS
_CHUNK_E = 2000
_NCHUNK = _EPW // _CHUNK_E
_WIN = 2176
_HROWS = 64
_NHSUB = 10
_GRP = 25


def _sc_partials(nodes_2d, edges_t, gi_t):
    mesh = plsc.VectorSubcoreMesh(core_axis_name="c", subcore_axis_name="s")

    @functools.partial(
        pl.kernel,
        out_type=jax.ShapeDtypeStruct((_NC * _NPAD,), jnp.float32),
        mesh=mesh,
        compiler_params=pltpu.CompilerParams(needs_layout_passes=False),
        scratch_types=[
            pltpu.VMEM((_NPAD,), jnp.float32),
            pltpu.VMEM((2 * _HROWS, _D_NODE), jnp.float32),
            pltpu.VMEM((8, _WIN), jnp.float32),
            pltpu.VMEM((8, _WIN), jnp.float32),
            pltpu.VMEM((2, _WIN), jnp.int32),
            pltpu.VMEM((2, _WIN), jnp.int32),
            pltpu.VMEM((_NPAD,), jnp.float32),
            pltpu.VMEM((_HROWS,), jnp.float32),
            pltpu.VMEM((_STRIPE,), jnp.float32),
            pltpu.VMEM((_STRIPE,), jnp.float32),
            pltpu.VMEM((_STRIPE,), jnp.float32),
            pltpu.VMEM_SHARED((_NPAD,), jnp.float32),
            pltpu.VMEM_SHARED((_NS * _NPAD,), jnp.float32),
            pltpu.SemaphoreType.DMA,
            pltpu.SemaphoreType.DMA,
            pltpu.SemaphoreType.DMA,
            pltpu.SemaphoreType.DMA,
            pltpu.SemaphoreType.DMA,
            pltpu.SemaphoreType.DMA,
        ],
    )
    def k(nodes_hbm, ef_hbm, gi_hbm, out_hbm,
          nodes_v, rows_v, ec0, ec1, gc0, gc1, acc_v, stage_v,
          red_v, tmp0_v, tmp1_v, nodes_sh, acc_sh,
          sem0, sem1, sem2, sem3, sem4, sem5):
        c = lax.axis_index("c")
        s = lax.axis_index("s")
        wid = s * _NC + c
        base = wid * _EPW

        iota = lax.iota(jnp.int32, _L)
        col0 = jnp.zeros((_L,), jnp.int32)
        row0 = col0
        row1 = jnp.ones((_L,), jnp.int32)

        ebufs = (ec0, ec1)
        gbufs = (gc0, gc1)
        esems = (sem0, sem1)
        gsems = (sem2, sem3)

        def win_start(ci):
            cbase = base + ci * _CHUNK_E
            w = jnp.minimum((cbase // 128) * 128, _N_EDGES - _WIN)
            w = pl.multiple_of(w, 128)
            return w, cbase - w

        ecps = [None, None]
        gcps = [None, None]
        w0, _ = win_start(0)
        ecps[0] = pltpu.async_copy(
            ef_hbm.at[pl.ds(0, 8), pl.ds(w0, _WIN)], ebufs[0], esems[0])
        gcps[0] = pltpu.async_copy(
            gi_hbm.at[:, pl.ds(w0, _WIN)], gbufs[0], gsems[0])

        base_row = s * _STRIPE
        rbufs = (rows_v.at[pl.ds(0, _HROWS)], rows_v.at[pl.ds(_HROWS, _HROWS)])
        rsems = (sem4, sem5)
        sts = [jnp.minimum(base_row + i * _HROWS, _N_NODES - _HROWS)
               for i in range(_NHSUB)]
        rcps = [None, None]
        rcps[0] = pltpu.async_copy(
            nodes_hbm.at[pl.ds(sts[0], _HROWS)], rbufs[0], rsems[0])
        for i in range(_NHSUB):
            if i + 1 < _NHSUB:
                nb = (i + 1) % 2
                rcps[nb] = pltpu.async_copy(
                    nodes_hbm.at[pl.ds(sts[i + 1], _HROWS)], rbufs[nb],
                    rsems[nb])
            rcps[i % 2].wait()
            buf = rbufs[i % 2]
            for j in range(_HROWS // _L):
                stage_v[pl.ds(j * _L, _L)] = plsc.load_gather(
                    buf, [j * _L + iota, col0])
            pltpu.sync_copy(stage_v, nodes_sh.at[pl.ds(sts[i], _HROWS)])

        zeros = jnp.zeros((_L,), jnp.float32)

        def zero_body(i, carry):
            acc_v[pl.ds(i * _L, _L)] = zeros
            return carry

        lax.fori_loop(0, _NPAD // _L, zero_body, 0)

        plsc.subcore_barrier()
        pltpu.sync_copy(nodes_sh, nodes_v)

        for ci in range(_NCHUNK):
            if ci + 1 < _NCHUNK:
                nb = (ci + 1) % 2
                wn, _ = win_start(ci + 1)
                ecps[nb] = pltpu.async_copy(
                    ef_hbm.at[pl.ds(0, 8), pl.ds(wn, _WIN)],
                    ebufs[nb], esems[nb])
                gcps[nb] = pltpu.async_copy(
                    gi_hbm.at[:, pl.ds(wn, _WIN)], gbufs[nb], gsems[nb])
            ecps[ci % 2].wait()
            gcps[ci % 2].wait()
            ebuf = ebufs[ci % 2]
            gbuf = gbufs[ci % 2]
            _, rj = win_start(ci)

            def body(j, carry, ebuf=ebuf, gbuf=gbuf, rj=rj):
                off0 = rj + j * (_GRP * _L)
                st_a = {}
                st_b = {}
                for g in range(_GRP + 4):
                    if g < _GRP:
                        pos = off0 + g * _L + iota
                        st_a[g] = (plsc.load_gather(gbuf, [row0, pos]),
                                   plsc.load_gather(gbuf, [row1, pos]),
                                   plsc.load_gather(ebuf, [row0, pos]))
                    if 0 <= g - 2 < _GRP:
                        s_idx, r_idx, e = st_a.pop(g - 2)
                        st_b[g - 2] = (plsc.load_gather(nodes_v, [s_idx]),
                                       plsc.load_gather(nodes_v, [r_idx]),
                                       r_idx, e)
                    if 0 <= g - 4 < _GRP:
                        ns, nr, r_idx, e = st_b.pop(g - 4)
                        plsc.addupdate_scatter(acc_v, [r_idx], (ns - nr) / e)
                return carry

            lax.fori_loop(0, _CHUNK_E // (_GRP * _L), body, 0)

        pltpu.sync_copy(acc_v, acc_sh.at[pl.ds(s * _NPAD, _NPAD)])
        plsc.subcore_barrier()

        stripe = s * _STRIPE
        tmps = (tmp0_v, tmp1_v)

        def rsum_body(i, carry, t=None):
            off = i * _L
            red_v[pl.ds(off, _L)] = (
                red_v[pl.ds(off, _L)] + tmps[t][pl.ds(off, _L)])
            return carry

        pltpu.sync_copy(acc_sh.at[pl.ds(stripe, _STRIPE)], red_v)
        acps = [None, None]
        acps[1] = pltpu.async_copy(
            acc_sh.at[pl.ds(_NPAD + stripe, _STRIPE)], tmps[1], rsems[1])
        for t in range(1, _NS):
            if t + 1 < _NS:
                nb = (t + 1) % 2
                acps[nb] = pltpu.async_copy(
                    acc_sh.at[pl.ds((t + 1) * _NPAD + stripe, _STRIPE)],
                    tmps[nb], rsems[nb])
            acps[t % 2].wait()
            lax.fori_loop(0, _STRIPE // _L,
                          functools.partial(rsum_body, t=t % 2), 0)

        pltpu.sync_copy(red_v, out_hbm.at[pl.ds(c * _NPAD + stripe, _STRIPE)])

    return k(nodes_2d, edges_t, gi_t)


def _tc_combine(partials):
    def body(p_ref, o_ref):
        a = p_ref[pl.ds(0, _NPAD)]
        b = p_ref[pl.ds(_NPAD, _NPAD)]
        o_ref[...] = (a + b)[: _N_NODES]

    return pl.pallas_call(
        body,
        out_shape=jax.ShapeDtypeStruct((_N_NODES,), jnp.float32),
    )(partials)


def kernel(input_node, input_edge, graph_index):
    partials = _sc_partials(input_node, input_edge.T, graph_index.T)
    return _tc_combine(partials)
